# baseline (speedup 1.0000x reference)
import functools

import jax
import jax.numpy as jnp
from jax import lax
from jax.experimental import pallas as pl
from jax.experimental.pallas import tpu as pltpu
from jax.experimental.pallas import tpu_sc as plsc

N = 10000
NP = 10240
E = 320000
F = 128
D = 64
HID = 50
B = 64
RATIO = 0.5
SNP = 3001
SNP_P = 3072
H1 = 512
CAT = 768
NCLS_P = 128

NC, NS = 2, 16
NT = NC * NS
ET = E // NT
CH = 80
NCH = ET // CH
RPT = NP // NS

_HIGH = lax.Precision.HIGHEST


def _dot(a, b):
    return jnp.dot(a, b, precision=_HIGH, preferred_element_type=jnp.float32)


def _mesh():
    return plsc.VectorSubcoreMesh(
        core_axis_name="c", subcore_axis_name="s",
        num_cores=NC, num_subcores=NS)


_SC_PARAMS = pltpu.CompilerParams(use_tc_tiling_on_sc=False,
                                  needs_layout_passes=False)


def _zero_acc(buf0, acc, s, d, ch):
    @pl.loop(0, ch)
    def _(i):
        for kk in range(d // 16):
            buf0[i, pl.ds(kk * 16, 16)] = jnp.zeros((16,), jnp.float32)
    row0 = pl.multiple_of(s * RPT, 8)
    for z in range(RPT // ch):
        pltpu.sync_copy(buf0, acc.at[pl.ds(row0 + z * ch, ch)])
    return row0


def _sc_hist(with_mask):
    scratch = [
        pltpu.VMEM((ET,), jnp.int32),
        pltpu.VMEM((NCH, CH), jnp.int32),
        pltpu.VMEM((NP,), jnp.float32),
        pltpu.VMEM((CH, 16), jnp.float32),
        pltpu.VMEM((CH, 16), jnp.float32),
        pltpu.VMEM_SHARED((NP, 16), jnp.float32),
        pltpu.SemaphoreType.DMA,
        pltpu.SemaphoreType.DMA,
    ]

    def body(*refs):
        if with_mask:
            (m_hbm, src_hbm, dst3_hbm, out_hbm,
             src_v, dst_v, m_v, buf0, buf1, acc, sems, sems2) = refs
        else:
            (dst3_hbm, out_hbm,
             src_v, dst_v, m_v, buf0, buf1, acc, sems, sems2) = refs
        c = lax.axis_index("c")
        s = lax.axis_index("s")
        tid = s * NC + c

        pltpu.sync_copy(dst3_hbm.at[tid], dst_v)
        if with_mask:
            pltpu.sync_copy(m_hbm, m_v)
            pltpu.sync_copy(src_hbm.at[pl.ds(pl.multiple_of(tid * ET, 8),
                                             ET)], src_v)
        row0 = _zero_acc(buf0, acc, s, 16, CH)
        plsc.subcore_barrier()

        zcol = jnp.zeros((16,), jnp.int32)
        lane = lax.iota(jnp.int32, 16)

        if with_mask:
            def fill(j, buf):
                for gi in range(CH // 16):
                    idx = src_v[pl.ds(pl.multiple_of(j * CH, 8) + gi * 16,
                                      16)]
                    vals = plsc.load_gather(m_v, [idx])
                    plsc.store_scatter(buf, [lane + gi * 16, zcol], vals)
        else:
            def fill(j, buf):
                del j
                for gi in range(CH // 16):
                    plsc.store_scatter(buf, [lane + gi * 16, zcol],
                                       jnp.ones((16,), jnp.float32))

        def scatter(j, buf, sem):
            return pltpu.async_copy(buf, acc.at[dst_v.at[j]], sem, add=True)

        fill(0, buf0)

        @pl.loop(0, NCH - 1, step=2)
        def _(j):
            d0 = scatter(j, buf0, sems)
            fill(j + 1, buf1)
            d1 = scatter(j + 1, buf1, sems2)
            d0.wait()
            fill(j + 2, buf0)
            d1.wait()

        scatter(NCH - 1, buf0, sems).wait()
        plsc.subcore_barrier()
        pltpu.sync_copy(acc.at[pl.ds(row0, RPT)],
                        out_hbm.at[c, pl.ds(row0, RPT)])

    return functools.partial(
        pl.kernel, body,
        out_type=jax.ShapeDtypeStruct((NC, NP, 16), jnp.float32),
        mesh=_mesh(), scratch_types=scratch, compiler_params=_SC_PARAMS)()


def _sc_hist_ones(dst3):
    return _sc_hist(False)(dst3)


def _sc_hist_mask(m, src, dst3):
    return _sc_hist(True)(m, src, dst3)


CHA = 128
NFA = 78
TOFF = NFA * CHA


def _sc_accum():
    scratch = [
        pltpu.VMEM((ET,), jnp.int32),
        pltpu.VMEM((NFA, CHA), jnp.int32),
        pltpu.VMEM((1, 16), jnp.int32),
        pltpu.VMEM((CHA, D), jnp.float32),
        pltpu.VMEM((CHA, D), jnp.float32),
        pltpu.VMEM_SHARED((NP, D), jnp.float32),
        pltpu.SemaphoreType.DMA,
        pltpu.SemaphoreType.DMA,
        pltpu.SemaphoreType.DMA,
    ]

    def body(g_hbm, src_hbm, dstA_hbm, dstT_hbm, out_hbm,
             src_v, dst_v, dstt_v, buf0, buf1, acc, semg, sems, sems2):
        c = lax.axis_index("c")
        s = lax.axis_index("s")
        tid = s * NC + c

        pltpu.sync_copy(dstA_hbm.at[tid], dst_v)
        pltpu.sync_copy(dstT_hbm.at[tid], dstt_v)
        pltpu.sync_copy(src_hbm.at[pl.ds(pl.multiple_of(tid * ET, 8), ET)],
                        src_v)
        row0 = _zero_acc(buf0, acc, s, D, CHA)
        plsc.subcore_barrier()

        def gather(j, buf):
            off = pl.multiple_of(j * CHA, 8)
            return pltpu.async_copy(
                g_hbm.at[src_v.at[pl.ds(off, CHA)]], buf, semg)

        def scatter(j, buf, sem):
            return pltpu.async_copy(buf, acc.at[dst_v.at[j]], sem, add=True)

        gather(0, buf0).wait()

        @pl.loop(0, NFA - 1, step=2)
        def _(j):
            dg = gather(j + 1, buf1)
            ds0 = scatter(j, buf0, sems)
            dg.wait()
            ds0.wait()
            ds1 = scatter(j + 1, buf1, sems2)

            @pl.when(j + 2 < NFA)
            def _():
                gather(j + 2, buf0).wait()
            ds1.wait()

        pltpu.async_copy(
            g_hbm.at[src_v.at[pl.ds(TOFF, 16)]],
            buf0.at[pl.ds(0, 16)], semg).wait()
        pltpu.async_copy(buf0.at[pl.ds(0, 16)],
                         acc.at[dstt_v.at[0]], sems, add=True).wait()

        plsc.subcore_barrier()
        pltpu.sync_copy(acc.at[pl.ds(row0, RPT)],
                        out_hbm.at[c, pl.ds(row0, RPT)])

    return functools.partial(
        pl.kernel, body,
        out_type=jax.ShapeDtypeStruct((NC, NP, D), jnp.float32),
        mesh=_mesh(), scratch_types=scratch, compiler_params=_SC_PARAMS)()


def _sc_gather_sum64(g, src, dstA, dstT):
    return _sc_accum()(g, src, dstA, dstT)


def _k1_prescale(xp, w1p, hist1):
    def body(x_ref, w_ref, h_ref, g_ref):
        raw = h_ref[0, :, 0:1] + h_ref[1, :, 0:1]
        dinv = lax.rsqrt(raw + 1.0)
        g_ref[...] = _dot(x_ref[...], w_ref[...]) * dinv

    return pl.pallas_call(
        body,
        out_shape=jax.ShapeDtypeStruct((NP, D), jnp.float32),
    )(xp, w1p, hist1)


def _k_layer_out(acc, g, hist, mcol, brow, prow, w_next):
    with_next = w_next is not None

    def body(*refs):
        if with_next:
            (a_ref, g_ref, h_ref, m_ref, b_ref, p_ref, wn_ref,
             xp_ref, sc_ref, hn_ref) = refs
        else:
            (a_ref, g_ref, h_ref, m_ref, b_ref, p_ref,
             xp_ref, sc_ref) = refs
        raw = h_ref[0, :, 0:1] + h_ref[1, :, 0:1]
        m = m_ref[...]
        dinv = jnp.where(m > 0, lax.rsqrt(raw + 1.0), 0.0)
        x = dinv * (a_ref[0] + a_ref[1] + g_ref[...]) + b_ref[...]
        p = p_ref[...]
        nrm = jnp.sqrt(jnp.sum(p * p))
        score = jnp.sum(x * p, axis=1, keepdims=True) / nrm
        sc_ref[...] = score
        xp = x * jnp.tanh(score)
        xp_ref[...] = xp
        if with_next:
            hn_ref[...] = _dot(xp, wn_ref[...])

    blk = 2048
    out_shape = [jax.ShapeDtypeStruct((NP, D), jnp.float32),
                 jax.ShapeDtypeStruct((NP, 1), jnp.float32)]
    out_specs = [pl.BlockSpec((blk, D), lambda i: (i, 0)),
                 pl.BlockSpec((blk, 1), lambda i: (i, 0))]
    in_specs = [pl.BlockSpec((NC, blk, D), lambda i: (0, i, 0)),
                pl.BlockSpec((blk, D), lambda i: (i, 0)),
                pl.BlockSpec((NC, blk, 16), lambda i: (0, i, 0)),
                pl.BlockSpec((blk, 1), lambda i: (i, 0)),
                pl.BlockSpec((1, D), lambda i: (0, 0)),
                pl.BlockSpec((1, D), lambda i: (0, 0))]
    args = [acc, g, hist, mcol, brow, prow]
    if with_next:
        out_shape.append(jax.ShapeDtypeStruct((NP, D), jnp.float32))
        out_specs.append(pl.BlockSpec((blk, D), lambda i: (i, 0)))
        in_specs.append(pl.BlockSpec((D, D), lambda i: (0, 0)))
        args.append(w_next)
    return pl.pallas_call(
        body, grid=(NP // blk,), in_specs=in_specs, out_specs=out_specs,
        out_shape=out_shape)(*args)


RB = 1024
CC = 512


def _k_rank(scol, bcol, mcol):
    def body(sc_r, bc_r, mc_r, mn_ref, cnt_s):
        g_row = lax.broadcasted_iota(jnp.int32, (1, B), 1).astype(jnp.float32)
        valid = jnp.sum(jnp.where(bc_r[...] == g_row, mc_r[...], 0.0),
                        axis=0, keepdims=True)
        k_row = jnp.ceil(RATIO * valid)

        def row_block(r, _):
            i0 = r * RB
            sc = sc_r[pl.ds(i0, RB), :]
            bc = bc_r[pl.ds(i0, RB), :]
            mc = mc_r[pl.ds(i0, RB), :]
            msc = jnp.where(mc > 0, sc, -jnp.inf)
            ig = lax.broadcasted_iota(jnp.int32, (RB, 1), 0) + i0
            kv = jnp.sum(jnp.where(bc == g_row, k_row, 0.0),
                         axis=1, keepdims=True)
            rbmin = jnp.min(jnp.where(bc < 0, jnp.inf, bc))
            rbmax = jnp.max(bc)
            cnt_s[...] = jnp.zeros((RB, 1), jnp.float32)

            def col_chunk(cj, _):
                c0 = cj * CC
                br = bc_r[pl.ds(c0, CC), :].T
                cmin = jnp.min(br)
                cmax = jnp.max(br)

                @pl.when((cmin <= rbmax) & (cmax >= rbmin))
                def _():
                    sr = sc_r[pl.ds(c0, CC), :].T
                    mr = mc_r[pl.ds(c0, CC), :].T
                    msr = jnp.where(mr > 0, sr, -jnp.inf)
                    jg = lax.broadcasted_iota(jnp.int32, (1, CC), 1) + c0
                    same = (bc == br)
                    better = (msr > msc) | ((msr == msc) & (jg < ig))
                    cnt_s[...] += jnp.sum(
                        jnp.where(same & better, 1.0, 0.0),
                        axis=1, keepdims=True)
                return 0

            lax.fori_loop(0, NP // CC, col_chunk, 0)
            mnew = jnp.where((mc > 0) & (cnt_s[...] < kv), 1.0, 0.0)
            mn_ref[pl.ds(i0, RB), :] = mnew
            return 0

        lax.fori_loop(0, NP // RB, row_block, 0)

    return pl.pallas_call(
        body,
        out_shape=jax.ShapeDtypeStruct((NP, 1), jnp.float32),
        scratch_shapes=[pltpu.VMEM((RB, 1), jnp.float32)],
    )(scol, bcol, mcol)


def _k_prescale2(hist2, mcol, h2):
    def body(h_ref, m_ref, x_ref, g_ref):
        raw = h_ref[0, :, 0:1] + h_ref[1, :, 0:1]
        dinv = jnp.where(m_ref[...] > 0, lax.rsqrt(raw + 1.0), 0.0)
        g_ref[...] = dinv * x_ref[...]

    return pl.pallas_call(
        body,
        out_shape=jax.ShapeDtypeStruct((NP, D), jnp.float32),
    )(hist2, mcol, h2)


def _k_head(x1p, x2p, m1c, m2c, bcol, xsnp, iw8,
            fc1wp, fc1bp, bn1gp, bn1bp, fc2wp, fc2bp,
            fc3arr, fc3bp, bn2gp, bn2bp, fc4wp, fc4bp):
    def body(x1_r, x2_r, m1c_r, m2c_r, bc_r,
             xs_r, iw_r, w1_r, b1_r, g1_r, be1_r, w2_r, b2_r,
             w3_r, b3_r, g2_r, be2_r, w4_r, b4_r,
             reg_ref, cls_ref, gmp1_s, gmp2_s):
        g_col = lax.broadcasted_iota(jnp.int32, (B, 1), 0).astype(jnp.float32)
        br = bc_r[...].T

        def gap(x_r, m_row):
            onehot = jnp.where((br == g_col) & (m_row > 0), 1.0, 0.0)
            ssum = _dot(onehot, x_r[...])
            cnt = jnp.sum(onehot, axis=1, keepdims=True)
            return ssum / jnp.maximum(cnt, 1.0)

        gap1 = gap(x1_r, m1c_r[...].T)
        gap2 = gap(x2_r, m2c_r[...].T)

        WGW = 1024

        def gmp(x_r, m_r, out_s):
            def one(g, _):
                gf = g.astype(jnp.float32)
                start = jnp.sum(jnp.where(br < gf, 1.0, 0.0))
                end = jnp.sum(jnp.where(br <= gf, 1.0, 0.0))
                a = jnp.minimum((start * 0.125).astype(jnp.int32) * 8,
                                NP - WGW)
                bw = bc_r[pl.ds(a, WGW), :]
                mw = m_r[pl.ds(a, WGW), :]
                xw = x_r[pl.ds(a, WGW), :]
                sel = (bw == gf) & (mw > 0)
                mx = jnp.max(jnp.where(sel, xw, -jnp.inf),
                             axis=0, keepdims=True)
                out_s[pl.ds(g, 1), :] = mx

                @pl.when(end > a.astype(jnp.float32) + WGW)
                def _():
                    self_f = (bc_r[...] == gf) & (m_r[...] > 0)
                    mxf = jnp.max(jnp.where(self_f, x_r[...], -jnp.inf),
                                  axis=0, keepdims=True)
                    out_s[pl.ds(g, 1), :] = mxf
                return 0
            lax.fori_loop(0, B, one, 0)

        gmp(x1_r, m1c_r, gmp1_s)
        gmp(x2_r, m2c_r, gmp2_s)

        iw_mean = jnp.mean(iw_r[...], axis=0, keepdims=True)
        snp0 = jnp.maximum(xs_r[...] * iw_mean, 0.0)
        t1 = _dot(snp0, w1_r[...]) + b1_r[...]
        mu1 = jnp.mean(t1, axis=0, keepdims=True)
        var1 = jnp.mean((t1 - mu1) ** 2, axis=0, keepdims=True)
        snp = jnp.maximum(
            g1_r[...] * (t1 - mu1) / jnp.sqrt(var1 + 1e-5) + be1_r[...], 0.0)

        reg_ref[...] = jnp.maximum(_dot(snp, w2_r[...]) + b2_r[...], 0.0)

        cat = jnp.concatenate(
            [snp, gmp1_s[...], gap1, gmp2_s[...], gap2], axis=1)
        t3 = _dot(cat, w3_r[...]) + b3_r[...]
        mu2 = jnp.mean(t3, axis=0, keepdims=True)
        var2 = jnp.mean((t3 - mu2) ** 2, axis=0, keepdims=True)
        xx2 = g2_r[...] * (t3 - mu2) / jnp.sqrt(var2 + 1e-5) + be2_r[...]

        logits = _dot(xx2, w4_r[...]) + b4_r[...]
        lmask = lax.broadcasted_iota(jnp.int32, (B, NCLS_P), 1) < 4
        mx = jnp.max(jnp.where(lmask, logits, -jnp.inf),
                     axis=1, keepdims=True)
        lse = jnp.log(jnp.sum(
            jnp.where(lmask, jnp.exp(logits - mx), 0.0),
            axis=1, keepdims=True)) + mx
        cls_ref[...] = logits - lse

    return pl.pallas_call(
        body,
        out_shape=[jax.ShapeDtypeStruct((B, NCLS_P), jnp.float32),
                   jax.ShapeDtypeStruct((B, NCLS_P), jnp.float32)],
        scratch_shapes=[pltpu.VMEM((B, D), jnp.float32),
                        pltpu.VMEM((B, D), jnp.float32)],
    )(x1p, x2p, m1c, m2c, bcol, xsnp, iw8,
      fc1wp, fc1bp, bn1gp, bn1bp, fc2wp, fc2bp,
      fc3arr, fc3bp, bn2gp, bn2bp, fc4wp, fc4bp)


def kernel(x, edge_index, batch, x_snp, W1, b1, W2, b2, pool1_w, pool2_w,
           input_w, fc1_w, fc1_b, bn1_g, bn1_b, fc2_w, fc2_b, fc3_w, fc3_b,
           bn2_g, bn2_b, fc4_w, fc4_b):
    f32 = jnp.float32
    src = edge_index[0]
    dst = edge_index[1]
    dst3 = dst.reshape(NT, NCH, CH)
    dst2 = dst.reshape(NT, ET)
    dstA = dst2[:, :TOFF].reshape(NT, NFA, CHA)
    dstT = dst2[:, TOFF:].reshape(NT, 1, 16)

    xp = jnp.pad(x, ((0, NP - N), (0, 0)))
    w1p = jnp.pad(W1, ((0, 0), (0, D - HID)))
    w2p = jnp.pad(W2, ((0, D - HID), (0, D - HID)))
    b1r = jnp.pad(b1, (0, D - HID)).reshape(1, D)
    b2r = jnp.pad(b2, (0, D - HID)).reshape(1, D)
    p1r = jnp.pad(pool1_w, (0, D - HID)).reshape(1, D)
    p2r = jnp.pad(pool2_w, (0, D - HID)).reshape(1, D)

    bcol = jnp.pad(batch.astype(f32), (0, NP - N),
                   constant_values=64.0).reshape(NP, 1)
    m0col = jnp.pad(jnp.ones((N,), f32), (0, NP - N)).reshape(NP, 1)

    hist1 = _sc_hist_ones(dst3)
    g1 = _k1_prescale(xp, w1p, hist1)
    acc1 = _sc_gather_sum64(g1, src, dstA, dstT)
    x1p, score1, h2 = _k_layer_out(acc1, g1, hist1, m0col, b1r, p1r, w2p)
    m1col = _k_rank(score1, bcol, m0col)

    hist2 = _sc_hist_mask(m1col.reshape(NP), src, dst3)
    g2 = _k_prescale2(hist2, m1col, h2)
    acc2 = _sc_gather_sum64(g2, src, dstA, dstT)
    x2p, score2 = _k_layer_out(acc2, g2, hist2, m1col, b2r, p2r, None)
    m2col = _k_rank(score2, bcol, m1col)

    xsnp = jnp.pad(x_snp, ((0, 0), (0, SNP_P - SNP)))
    iw8 = jnp.pad(input_w.reshape(8, SNP), ((0, 0), (0, SNP_P - SNP)))
    fc1wp = jnp.pad(fc1_w, ((0, SNP_P - SNP), (0, H1 - 500)))
    fc1bp = jnp.pad(fc1_b, (0, H1 - 500)).reshape(1, H1)
    bn1gp = jnp.pad(bn1_g, (0, H1 - 500)).reshape(1, H1)
    bn1bp = jnp.pad(bn1_b, (0, H1 - 500)).reshape(1, H1)
    fc2wp = jnp.pad(fc2_w, ((0, H1 - 500), (0, NCLS_P - 8)))
    fc2bp = jnp.pad(fc2_b, (0, NCLS_P - 8)).reshape(1, NCLS_P)
    fc3arr = jnp.zeros((CAT, D), f32)
    fc3arr = fc3arr.at[0:500, 0:HID].set(fc3_w[0:500])
    for t in range(4):
        fc3arr = fc3arr.at[H1 + D * t: H1 + D * t + HID, 0:HID].set(
            fc3_w[500 + HID * t: 500 + HID * (t + 1)])
    fc3bp = jnp.pad(fc3_b, (0, D - HID)).reshape(1, D)
    bn2gp = jnp.pad(bn2_g, (0, D - HID)).reshape(1, D)
    bn2bp = jnp.pad(bn2_b, (0, D - HID)).reshape(1, D)
    fc4wp = jnp.pad(fc4_w, ((0, D - HID), (0, NCLS_P - 4)))
    fc4bp = jnp.pad(fc4_b, (0, NCLS_P - 4)).reshape(1, NCLS_P)

    reg, cls = _k_head(x1p, x2p, m1col, m2col, bcol,
                       xsnp, iw8, fc1wp, fc1bp, bn1gp, bn1bp, fc2wp, fc2bp,
                       fc3arr, fc3bp, bn2gp, bn2bp, fc4wp, fc4bp)
    return (reg[:, :8], cls[:, :4])


# speedup vs baseline: 1.0664x; 1.0664x over previous
import functools

import jax
import jax.numpy as jnp
from jax import lax
from jax.experimental import pallas as pl
from jax.experimental.pallas import tpu as pltpu
from jax.experimental.pallas import tpu_sc as plsc

N = 10000
NP = 10240
E = 320000
F = 128
D = 64
HID = 50
B = 64
RATIO = 0.5
SNP = 3001
SNP_P = 3072
H1 = 512
CAT = 768
NCLS_P = 128

NC, NS = 2, 16
NT = NC * NS
ET = E // NT
CH = 80
NCH = ET // CH
RPT = NP // NS

_HIGH = lax.Precision.HIGHEST


def _dot(a, b):
    return jnp.dot(a, b, precision=_HIGH, preferred_element_type=jnp.float32)


def _mesh():
    return plsc.VectorSubcoreMesh(
        core_axis_name="c", subcore_axis_name="s",
        num_cores=NC, num_subcores=NS)


_SC_PARAMS = pltpu.CompilerParams(use_tc_tiling_on_sc=False,
                                  needs_layout_passes=False)


def _zero_acc(buf0, acc, s, d, ch):
    @pl.loop(0, ch)
    def _(i):
        for kk in range(d // 16):
            buf0[i, pl.ds(kk * 16, 16)] = jnp.zeros((16,), jnp.float32)
    row0 = pl.multiple_of(s * RPT, 8)
    for z in range(RPT // ch):
        pltpu.sync_copy(buf0, acc.at[pl.ds(row0 + z * ch, ch)])
    return row0


def _sc_hist(with_mask):
    scratch = [
        pltpu.VMEM((ET,), jnp.int32),
        pltpu.VMEM((NCH, CH), jnp.int32),
        pltpu.VMEM((NP,), jnp.float32),
        pltpu.VMEM((CH, 16), jnp.float32),
        pltpu.VMEM((CH, 16), jnp.float32),
        pltpu.VMEM_SHARED((NP, 16), jnp.float32),
        pltpu.SemaphoreType.DMA,
        pltpu.SemaphoreType.DMA,
    ]

    def body(*refs):
        if with_mask:
            (m_hbm, src_hbm, dst3_hbm, out_hbm,
             src_v, dst_v, m_v, buf0, buf1, acc, sems, sems2) = refs
        else:
            (dst3_hbm, out_hbm,
             src_v, dst_v, m_v, buf0, buf1, acc, sems, sems2) = refs
        c = lax.axis_index("c")
        s = lax.axis_index("s")
        tid = s * NC + c

        pltpu.sync_copy(dst3_hbm.at[tid], dst_v)
        if with_mask:
            pltpu.sync_copy(m_hbm, m_v)
            pltpu.sync_copy(src_hbm.at[pl.ds(pl.multiple_of(tid * ET, 8),
                                             ET)], src_v)
        row0 = _zero_acc(buf0, acc, s, 16, CH)
        plsc.subcore_barrier()

        zcol = jnp.zeros((16,), jnp.int32)
        lane = lax.iota(jnp.int32, 16)

        if with_mask:
            def fill(j, buf):
                for gi in range(CH // 16):
                    idx = src_v[pl.ds(pl.multiple_of(j * CH, 8) + gi * 16,
                                      16)]
                    vals = plsc.load_gather(m_v, [idx])
                    plsc.store_scatter(buf, [lane + gi * 16, zcol], vals)
        else:
            def fill(j, buf):
                del j
                for gi in range(CH // 16):
                    plsc.store_scatter(buf, [lane + gi * 16, zcol],
                                       jnp.ones((16,), jnp.float32))

        def scatter(j, buf, sem):
            return pltpu.async_copy(buf, acc.at[dst_v.at[j]], sem, add=True)

        fill(0, buf0)

        @pl.loop(0, NCH - 1, step=2)
        def _(j):
            d0 = scatter(j, buf0, sems)
            fill(j + 1, buf1)
            d1 = scatter(j + 1, buf1, sems2)
            d0.wait()
            fill(j + 2, buf0)
            d1.wait()

        scatter(NCH - 1, buf0, sems).wait()
        plsc.subcore_barrier()
        pltpu.sync_copy(acc.at[pl.ds(row0, RPT)],
                        out_hbm.at[c, pl.ds(row0, RPT)])

    return functools.partial(
        pl.kernel, body,
        out_type=jax.ShapeDtypeStruct((NC, NP, 16), jnp.float32),
        mesh=_mesh(), scratch_types=scratch, compiler_params=_SC_PARAMS)()


def _sc_hist_ones(dst3):
    return _sc_hist(False)(dst3)


def _sc_hist_mask(m, src, dst3):
    return _sc_hist(True)(m, src, dst3)


CHA = 128
NFA = 78
TOFF = NFA * CHA


def _sc_accum():
    scratch = [
        pltpu.VMEM((ET,), jnp.int32),
        pltpu.VMEM((NFA, CHA), jnp.int32),
        pltpu.VMEM((1, 16), jnp.int32),
        pltpu.VMEM((CHA, D), jnp.float32),
        pltpu.VMEM((CHA, D), jnp.float32),
        pltpu.VMEM_SHARED((NP, D), jnp.float32),
        pltpu.SemaphoreType.DMA,
        pltpu.SemaphoreType.DMA,
        pltpu.SemaphoreType.DMA,
    ]

    def body(g_hbm, src_hbm, dstA_hbm, dstT_hbm, out_hbm,
             src_v, dst_v, dstt_v, buf0, buf1, acc, semg, sems, sems2):
        c = lax.axis_index("c")
        s = lax.axis_index("s")
        tid = s * NC + c

        pltpu.sync_copy(dstA_hbm.at[tid], dst_v)
        pltpu.sync_copy(dstT_hbm.at[tid], dstt_v)
        pltpu.sync_copy(src_hbm.at[pl.ds(pl.multiple_of(tid * ET, 8), ET)],
                        src_v)
        row0 = _zero_acc(buf0, acc, s, D, CHA)
        plsc.subcore_barrier()

        def gather(j, buf):
            off = pl.multiple_of(j * CHA, 8)
            return pltpu.async_copy(
                g_hbm.at[src_v.at[pl.ds(off, CHA)]], buf, semg)

        def scatter(j, buf, sem):
            return pltpu.async_copy(buf, acc.at[dst_v.at[j]], sem, add=True)

        gather(0, buf0).wait()

        @pl.loop(0, NFA - 1, step=2)
        def _(j):
            dg = gather(j + 1, buf1)
            ds0 = scatter(j, buf0, sems)
            dg.wait()
            ds0.wait()
            ds1 = scatter(j + 1, buf1, sems2)

            @pl.when(j + 2 < NFA)
            def _():
                gather(j + 2, buf0).wait()
            ds1.wait()

        pltpu.async_copy(
            g_hbm.at[src_v.at[pl.ds(TOFF, 16)]],
            buf0.at[pl.ds(0, 16)], semg).wait()
        pltpu.async_copy(buf0.at[pl.ds(0, 16)],
                         acc.at[dstt_v.at[0]], sems, add=True).wait()

        plsc.subcore_barrier()
        pltpu.sync_copy(acc.at[pl.ds(row0, RPT)],
                        out_hbm.at[c, pl.ds(row0, RPT)])

    return functools.partial(
        pl.kernel, body,
        out_type=jax.ShapeDtypeStruct((NC, NP, D), jnp.float32),
        mesh=_mesh(), scratch_types=scratch, compiler_params=_SC_PARAMS)()


def _sc_gather_sum64(g, src, dstA, dstT):
    return _sc_accum()(g, src, dstA, dstT)


def _k1_prescale(xp, w1p, hist1):
    def body(x_ref, w_ref, h_ref, g_ref):
        raw = h_ref[0, :, 0:1] + h_ref[1, :, 0:1]
        dinv = lax.rsqrt(raw + 1.0)
        g_ref[...] = _dot(x_ref[...], w_ref[...]) * dinv

    return pl.pallas_call(
        body,
        out_shape=jax.ShapeDtypeStruct((NP, D), jnp.float32),
    )(xp, w1p, hist1)


def _k_layer_out(acc, g, hist, mcol, brow, prow, w_next):
    with_next = w_next is not None

    def body(*refs):
        if with_next:
            (a_ref, g_ref, h_ref, m_ref, b_ref, p_ref, wn_ref,
             xp_ref, sc_ref, hn_ref) = refs
        else:
            (a_ref, g_ref, h_ref, m_ref, b_ref, p_ref,
             xp_ref, sc_ref) = refs
        raw = h_ref[0, :, 0:1] + h_ref[1, :, 0:1]
        m = m_ref[...]
        dinv = jnp.where(m > 0, lax.rsqrt(raw + 1.0), 0.0)
        x = dinv * (a_ref[0] + a_ref[1] + g_ref[...]) + b_ref[...]
        p = p_ref[...]
        nrm = jnp.sqrt(jnp.sum(p * p))
        score = jnp.sum(x * p, axis=1, keepdims=True) / nrm
        sc_ref[...] = score
        xp = x * jnp.tanh(score)
        xp_ref[...] = xp
        if with_next:
            hn_ref[...] = _dot(xp, wn_ref[...])

    blk = 2048
    out_shape = [jax.ShapeDtypeStruct((NP, D), jnp.float32),
                 jax.ShapeDtypeStruct((NP, 1), jnp.float32)]
    out_specs = [pl.BlockSpec((blk, D), lambda i: (i, 0)),
                 pl.BlockSpec((blk, 1), lambda i: (i, 0))]
    in_specs = [pl.BlockSpec((NC, blk, D), lambda i: (0, i, 0)),
                pl.BlockSpec((blk, D), lambda i: (i, 0)),
                pl.BlockSpec((NC, blk, 16), lambda i: (0, i, 0)),
                pl.BlockSpec((blk, 1), lambda i: (i, 0)),
                pl.BlockSpec((1, D), lambda i: (0, 0)),
                pl.BlockSpec((1, D), lambda i: (0, 0))]
    args = [acc, g, hist, mcol, brow, prow]
    if with_next:
        out_shape.append(jax.ShapeDtypeStruct((NP, D), jnp.float32))
        out_specs.append(pl.BlockSpec((blk, D), lambda i: (i, 0)))
        in_specs.append(pl.BlockSpec((D, D), lambda i: (0, 0)))
        args.append(w_next)
    return pl.pallas_call(
        body, grid=(NP // blk,), in_specs=in_specs, out_specs=out_specs,
        out_shape=out_shape)(*args)


RB = 1024
CC = 1024


def _k_rank(scol, srow, bcol, brow, mcol, mrow):
    def body(sc_r, sr_r, bc_r, br_r, mc_r, mr_r, mn_ref, cnt_s):
        g_row = lax.broadcasted_iota(jnp.int32, (1, B), 1).astype(jnp.float32)
        valid = jnp.sum(jnp.where(bc_r[...] == g_row, mc_r[...], 0.0),
                        axis=0, keepdims=True)
        k_row = jnp.ceil(RATIO * valid)

        def row_block(r, _):
            i0 = r * RB
            sc = sc_r[pl.ds(i0, RB), :]
            bc = bc_r[pl.ds(i0, RB), :]
            mc = mc_r[pl.ds(i0, RB), :]
            msc = jnp.where(mc > 0, sc, -jnp.inf)
            ig = lax.broadcasted_iota(jnp.int32, (RB, 1), 0) + i0
            kv = jnp.sum(jnp.where(bc == g_row, k_row, 0.0),
                         axis=1, keepdims=True)
            rbmin = jnp.min(jnp.where(bc < 0, jnp.inf, bc))
            rbmax = jnp.max(bc)
            cnt_s[...] = jnp.zeros((RB, 1), jnp.float32)

            def col_chunk(cj, _):
                c0 = cj * CC
                br = br_r[:, pl.ds(c0, CC)]
                cmin = jnp.min(br)
                cmax = jnp.max(br)

                @pl.when((cmin <= rbmax) & (cmax >= rbmin))
                def _():
                    sr = sr_r[:, pl.ds(c0, CC)]
                    mr = mr_r[:, pl.ds(c0, CC)]
                    msr = jnp.where(mr > 0, sr, -jnp.inf)
                    jg = lax.broadcasted_iota(jnp.int32, (1, CC), 1) + c0
                    same = (bc == br)
                    better = (msr > msc) | ((msr == msc) & (jg < ig))
                    cnt_s[...] += jnp.sum(
                        jnp.where(same & better, 1.0, 0.0),
                        axis=1, keepdims=True)
                return 0

            lax.fori_loop(0, NP // CC, col_chunk, 0)
            mnew = jnp.where((mc > 0) & (cnt_s[...] < kv), 1.0, 0.0)
            mn_ref[pl.ds(i0, RB), :] = mnew
            return 0

        lax.fori_loop(0, NP // RB, row_block, 0)

    return pl.pallas_call(
        body,
        out_shape=jax.ShapeDtypeStruct((NP, 1), jnp.float32),
        scratch_shapes=[pltpu.VMEM((RB, 1), jnp.float32)],
    )(scol, srow, bcol, brow, mcol, mrow)


def _k_prescale2(hist2, mcol, h2):
    def body(h_ref, m_ref, x_ref, g_ref):
        raw = h_ref[0, :, 0:1] + h_ref[1, :, 0:1]
        dinv = jnp.where(m_ref[...] > 0, lax.rsqrt(raw + 1.0), 0.0)
        g_ref[...] = dinv * x_ref[...]

    return pl.pallas_call(
        body,
        out_shape=jax.ShapeDtypeStruct((NP, D), jnp.float32),
    )(hist2, mcol, h2)


def _k_head(x1p, x2p, m1c, m2c, m1r, m2r, bcol, brow, xsnp, iw8,
            fc1wp, fc1bp, bn1gp, bn1bp, fc2wp, fc2bp,
            fc3arr, fc3bp, bn2gp, bn2bp, fc4wp, fc4bp):
    def body(x1_r, x2_r, m1c_r, m2c_r, m1r_r, m2r_r, bc_r, br_r,
             xs_r, iw_r, w1_r, b1_r, g1_r, be1_r, w2_r, b2_r,
             w3_r, b3_r, g2_r, be2_r, w4_r, b4_r,
             reg_ref, cls_ref, gmp1_s, gmp2_s):
        g_col = lax.broadcasted_iota(jnp.int32, (B, 1), 0).astype(jnp.float32)
        br = br_r[...]

        def gap(x_r, m_row):
            onehot = jnp.where((br == g_col) & (m_row > 0), 1.0, 0.0)
            ssum = _dot(onehot, x_r[...])
            cnt = jnp.sum(onehot, axis=1, keepdims=True)
            return ssum / jnp.maximum(cnt, 1.0)

        gap1 = gap(x1_r, m1r_r[...])
        gap2 = gap(x2_r, m2r_r[...])

        WGW = 1024

        def gmp(x_r, m_r, out_s):
            def one(g, _):
                gf = g.astype(jnp.float32)
                start = jnp.sum(jnp.where(br < gf, 1.0, 0.0))
                end = jnp.sum(jnp.where(br <= gf, 1.0, 0.0))
                a = jnp.minimum((start * 0.125).astype(jnp.int32) * 8,
                                NP - WGW)
                bw = bc_r[pl.ds(a, WGW), :]
                mw = m_r[pl.ds(a, WGW), :]
                xw = x_r[pl.ds(a, WGW), :]
                sel = (bw == gf) & (mw > 0)
                mx = jnp.max(jnp.where(sel, xw, -jnp.inf),
                             axis=0, keepdims=True)
                out_s[pl.ds(g, 1), :] = mx

                @pl.when(end > a.astype(jnp.float32) + WGW)
                def _():
                    self_f = (bc_r[...] == gf) & (m_r[...] > 0)
                    mxf = jnp.max(jnp.where(self_f, x_r[...], -jnp.inf),
                                  axis=0, keepdims=True)
                    out_s[pl.ds(g, 1), :] = mxf
                return 0
            lax.fori_loop(0, B, one, 0)

        gmp(x1_r, m1c_r, gmp1_s)
        gmp(x2_r, m2c_r, gmp2_s)

        iw_mean = jnp.mean(iw_r[...], axis=0, keepdims=True)
        snp0 = jnp.maximum(xs_r[...] * iw_mean, 0.0)
        t1 = _dot(snp0, w1_r[...]) + b1_r[...]
        mu1 = jnp.mean(t1, axis=0, keepdims=True)
        var1 = jnp.mean((t1 - mu1) ** 2, axis=0, keepdims=True)
        snp = jnp.maximum(
            g1_r[...] * (t1 - mu1) / jnp.sqrt(var1 + 1e-5) + be1_r[...], 0.0)

        reg_ref[...] = jnp.maximum(_dot(snp, w2_r[...]) + b2_r[...], 0.0)

        cat = jnp.concatenate(
            [snp, gmp1_s[...], gap1, gmp2_s[...], gap2], axis=1)
        t3 = _dot(cat, w3_r[...]) + b3_r[...]
        mu2 = jnp.mean(t3, axis=0, keepdims=True)
        var2 = jnp.mean((t3 - mu2) ** 2, axis=0, keepdims=True)
        xx2 = g2_r[...] * (t3 - mu2) / jnp.sqrt(var2 + 1e-5) + be2_r[...]

        logits = _dot(xx2, w4_r[...]) + b4_r[...]
        lmask = lax.broadcasted_iota(jnp.int32, (B, NCLS_P), 1) < 4
        mx = jnp.max(jnp.where(lmask, logits, -jnp.inf),
                     axis=1, keepdims=True)
        lse = jnp.log(jnp.sum(
            jnp.where(lmask, jnp.exp(logits - mx), 0.0),
            axis=1, keepdims=True)) + mx
        cls_ref[...] = logits - lse

    return pl.pallas_call(
        body,
        out_shape=[jax.ShapeDtypeStruct((B, NCLS_P), jnp.float32),
                   jax.ShapeDtypeStruct((B, NCLS_P), jnp.float32)],
        scratch_shapes=[pltpu.VMEM((B, D), jnp.float32),
                        pltpu.VMEM((B, D), jnp.float32)],
    )(x1p, x2p, m1c, m2c, m1r, m2r, bcol, brow, xsnp, iw8,
      fc1wp, fc1bp, bn1gp, bn1bp, fc2wp, fc2bp,
      fc3arr, fc3bp, bn2gp, bn2bp, fc4wp, fc4bp)


def kernel(x, edge_index, batch, x_snp, W1, b1, W2, b2, pool1_w, pool2_w,
           input_w, fc1_w, fc1_b, bn1_g, bn1_b, fc2_w, fc2_b, fc3_w, fc3_b,
           bn2_g, bn2_b, fc4_w, fc4_b):
    f32 = jnp.float32
    src = edge_index[0]
    dst = edge_index[1]
    dst3 = dst.reshape(NT, NCH, CH)
    dst2 = dst.reshape(NT, ET)
    dstA = dst2[:, :TOFF].reshape(NT, NFA, CHA)
    dstT = dst2[:, TOFF:].reshape(NT, 1, 16)

    xp = jnp.pad(x, ((0, NP - N), (0, 0)))
    w1p = jnp.pad(W1, ((0, 0), (0, D - HID)))
    w2p = jnp.pad(W2, ((0, D - HID), (0, D - HID)))
    b1r = jnp.pad(b1, (0, D - HID)).reshape(1, D)
    b2r = jnp.pad(b2, (0, D - HID)).reshape(1, D)
    p1r = jnp.pad(pool1_w, (0, D - HID)).reshape(1, D)
    p2r = jnp.pad(pool2_w, (0, D - HID)).reshape(1, D)

    bcol = jnp.pad(batch.astype(f32), (0, NP - N),
                   constant_values=64.0).reshape(NP, 1)
    brow = bcol.reshape(1, NP)
    m0col = jnp.pad(jnp.ones((N,), f32), (0, NP - N)).reshape(NP, 1)
    m0row = m0col.reshape(1, NP)

    hist1 = _sc_hist_ones(dst3)
    g1 = _k1_prescale(xp, w1p, hist1)
    acc1 = _sc_gather_sum64(g1, src, dstA, dstT)
    x1p, score1, h2 = _k_layer_out(acc1, g1, hist1, m0col, b1r, p1r, w2p)
    m1col = _k_rank(score1, score1.reshape(1, NP),
                    bcol, brow, m0col, m0row)
    m1row = m1col.reshape(1, NP)

    hist2 = _sc_hist_mask(m1col.reshape(NP), src, dst3)
    g2 = _k_prescale2(hist2, m1col, h2)
    acc2 = _sc_gather_sum64(g2, src, dstA, dstT)
    x2p, score2 = _k_layer_out(acc2, g2, hist2, m1col, b2r, p2r, None)
    m2col = _k_rank(score2, score2.reshape(1, NP),
                    bcol, brow, m1col, m1row)
    m2row = m2col.reshape(1, NP)

    xsnp = jnp.pad(x_snp, ((0, 0), (0, SNP_P - SNP)))
    iw8 = jnp.pad(input_w.reshape(8, SNP), ((0, 0), (0, SNP_P - SNP)))
    fc1wp = jnp.pad(fc1_w, ((0, SNP_P - SNP), (0, H1 - 500)))
    fc1bp = jnp.pad(fc1_b, (0, H1 - 500)).reshape(1, H1)
    bn1gp = jnp.pad(bn1_g, (0, H1 - 500)).reshape(1, H1)
    bn1bp = jnp.pad(bn1_b, (0, H1 - 500)).reshape(1, H1)
    fc2wp = jnp.pad(fc2_w, ((0, H1 - 500), (0, NCLS_P - 8)))
    fc2bp = jnp.pad(fc2_b, (0, NCLS_P - 8)).reshape(1, NCLS_P)
    fc3arr = jnp.zeros((CAT, D), f32)
    fc3arr = fc3arr.at[0:500, 0:HID].set(fc3_w[0:500])
    for t in range(4):
        fc3arr = fc3arr.at[H1 + D * t: H1 + D * t + HID, 0:HID].set(
            fc3_w[500 + HID * t: 500 + HID * (t + 1)])
    fc3bp = jnp.pad(fc3_b, (0, D - HID)).reshape(1, D)
    bn2gp = jnp.pad(bn2_g, (0, D - HID)).reshape(1, D)
    bn2bp = jnp.pad(bn2_b, (0, D - HID)).reshape(1, D)
    fc4wp = jnp.pad(fc4_w, ((0, D - HID), (0, NCLS_P - 4)))
    fc4bp = jnp.pad(fc4_b, (0, NCLS_P - 4)).reshape(1, NCLS_P)

    reg, cls = _k_head(x1p, x2p, m1col, m2col, m1row, m2row, bcol, brow,
                       xsnp, iw8, fc1wp, fc1bp, bn1gp, bn1bp, fc2wp, fc2bp,
                       fc3arr, fc3bp, bn2gp, bn2bp, fc4wp, fc4bp)
    return (reg[:, :8], cls[:, :4])
